# baseline (speedup 1.0000x reference)
import functools

import jax
import jax.numpy as jnp
from jax import lax
from jax.experimental import pallas as pl
from jax.experimental.pallas import tpu as pltpu
from jax.experimental.pallas import tpu_sc as plsc

NC = 2
NS = 16
NW = NC * NS
CHUNK = 128

N_NODES = 10000
N_EDGES = 320000
D = 128

TOTCH = N_EDGES // CHUNK
TOTCH_PAD = 2560
NPAD = 10240
RPT = NPAD // NS
ACC_CH = NPAD // CHUNK
ACC_CPT = ACC_CH // NS

T0_CHUNKS = 1664
CPT0 = T0_CHUNKS // NS
CPT1 = 56

IBLK = 8

DSLAB = 80

_sc_mesh = plsc.VectorSubcoreMesh(core_axis_name="c", subcore_axis_name="s")


def _deg_body(didx_hbm, out_hbm, didx_v, ones_v, zbuf_v, deg_sh, sem):
    cid = lax.axis_index("c")
    sid = lax.axis_index("s")
    wid = sid * NC + cid
    base = wid * DSLAB
    nch = jnp.minimum(DSLAB, TOTCH - base)

    def fill16(i, _):
        ones_v[pl.ds(i * 16, 16)] = jnp.full((16,), 1.0, jnp.float32)
        return ()

    lax.fori_loop(0, CHUNK // 16, fill16, ())

    def zfill(i, _):
        zbuf_v[pl.ds(i * 16, 16)] = jnp.zeros((16,), jnp.float32)
        return ()

    lax.fori_loop(0, RPT // 16, zfill, ())

    pltpu.sync_copy(zbuf_v, deg_sh.at[pl.ds(sid * RPT, RPT)])

    pltpu.sync_copy(didx_hbm.at[pl.ds(base, DSLAB)], didx_v)
    plsc.subcore_barrier()

    def fire(j, _):
        pltpu.async_copy(ones_v, deg_sh.at[didx_v.at[j]], sem, add=True)
        return ()

    lax.fori_loop(0, nch, fire, ())

    def drain(j, _):
        pltpu.make_async_copy(ones_v, deg_sh.at[didx_v.at[0]], sem).wait()
        return ()

    lax.fori_loop(0, nch, drain, ())
    plsc.subcore_barrier()

    pltpu.sync_copy(deg_sh.at[pl.ds(sid * RPT, RPT)], zbuf_v)
    pltpu.sync_copy(zbuf_v, out_hbm.at[cid, pl.ds(sid * RPT, RPT)])


_deg_kernel = functools.partial(
    pl.kernel,
    out_type=jax.ShapeDtypeStruct((NC, NPAD), jnp.float32),
    mesh=_sc_mesh,
    scratch_types=[
        pltpu.VMEM((DSLAB, CHUNK), jnp.int32),
        pltpu.VMEM((CHUNK,), jnp.float32),
        pltpu.VMEM((RPT,), jnp.float32),
        pltpu.VMEM_SHARED((NPAD,), jnp.float32),
        pltpu.SemaphoreType.DMA,
    ],
)(_deg_body)


def _scatter_body(g_hbm, sidx_hbm, didx_hbm, out_hbm,
                  sidx_v, didx_v, bufs, acc_sh, semg, sems, semi):
    cid = lax.axis_index("c")
    sid = lax.axis_index("s")
    base = jnp.where(cid == 0, sid * CPT0, T0_CHUNKS + sid * CPT1)
    nch = jnp.where(
        cid == 0,
        CPT0,
        jnp.clip(TOTCH - base, 0, CPT1),
    )

    def zfill(i, _):
        r = i // (D // 16)
        c = i % (D // 16)
        bufs[0, r, pl.ds(c * 16, 16)] = jnp.zeros((16,), jnp.float32)
        return ()

    lax.fori_loop(0, CHUNK * (D // 16), zfill, ())

    for m in range(ACC_CPT):
        ch = sid + NS * m
        pltpu.sync_copy(bufs.at[0], acc_sh.at[pl.ds(ch * CHUNK, CHUNK)])

    pltpu.sync_copy(sidx_hbm.at[pl.ds(base, IBLK)], sidx_v.at[0])
    pltpu.sync_copy(didx_hbm.at[pl.ds(base, IBLK)], didx_v.at[0])
    plsc.subcore_barrier()

    @pl.when(nch > 0)
    def _():
        pltpu.async_copy(g_hbm.at[sidx_v.at[0, 0]], bufs.at[0], semg.at[0])

    def step(j, _):
        par = j % 2
        npar = (j + 1) % 2
        blk = j // IBLK
        pos = j % IBLK

        @pl.when(j >= 1)
        def _():
            pltpu.make_async_copy(
                bufs.at[0], acc_sh.at[pl.ds(0, CHUNK)], sems.at[npar]
            ).wait()

        @pl.when((pos == 1) & ((blk + 1) * IBLK < nch))
        def _():
            pltpu.async_copy(
                sidx_hbm.at[pl.ds(base + (blk + 1) * IBLK, IBLK)],
                sidx_v.at[(blk + 1) % 2], semi)
            pltpu.async_copy(
                didx_hbm.at[pl.ds(base + (blk + 1) * IBLK, IBLK)],
                didx_v.at[(blk + 1) % 2], semi)

        @pl.when(j + 1 < nch)
        def _():
            @pl.when(pos == IBLK - 1)
            def _():
                pltpu.make_async_copy(
                    sidx_hbm.at[pl.ds(base, IBLK)], sidx_v.at[0], semi
                ).wait()
                pltpu.make_async_copy(
                    sidx_hbm.at[pl.ds(base, IBLK)], sidx_v.at[0], semi
                ).wait()

            pltpu.async_copy(
                g_hbm.at[sidx_v.at[((j + 1) // IBLK) % 2, (j + 1) % IBLK]],
                bufs.at[npar], semg.at[npar])

        pltpu.make_async_copy(
            g_hbm.at[sidx_v.at[0, 0]], bufs.at[par], semg.at[par]
        ).wait()
        pltpu.async_copy(
            bufs.at[par], acc_sh.at[didx_v.at[blk % 2, pos]],
            sems.at[par], add=True)
        return ()

    lax.fori_loop(0, nch, step, ())

    @pl.when(nch > 0)
    def _():
        pltpu.make_async_copy(
            bufs.at[0], acc_sh.at[pl.ds(0, CHUNK)], sems.at[(nch - 1) % 2]
        ).wait()

    plsc.subcore_barrier()

    for m in range(ACC_CPT):
        ch = sid + NS * m
        pltpu.sync_copy(acc_sh.at[pl.ds(ch * CHUNK, CHUNK)], bufs.at[1])
        pltpu.sync_copy(bufs.at[1], out_hbm.at[cid, pl.ds(ch * CHUNK, CHUNK)])


_scatter_kernel = functools.partial(
    pl.kernel,
    out_type=jax.ShapeDtypeStruct((NC, NPAD, D), jnp.float32),
    mesh=_sc_mesh,
    scratch_types=[
        pltpu.VMEM((2, IBLK, CHUNK), jnp.int32),
        pltpu.VMEM((2, IBLK, CHUNK), jnp.int32),
        pltpu.VMEM((2, CHUNK, D), jnp.float32),
        pltpu.VMEM_SHARED((NPAD, D), jnp.float32),
        pltpu.SemaphoreType.DMA((2,)),
        pltpu.SemaphoreType.DMA((2,)),
        pltpu.SemaphoreType.DMA,
    ],
)(_scatter_body)


BLK = 1024


def _lin_body(x_ref, w_ref, degp_ref, g_ref):
    deg = degp_ref[0, :] + degp_ref[1, :] + 1.0
    dis = lax.rsqrt(deg)
    h = lax.dot_general(
        x_ref[...], w_ref[...],
        (((1,), (1,)), ((), ())),
        preferred_element_type=jnp.float32,
    )
    g_ref[...] = h * dis[:, None]


def _final_body(accp_ref, g_ref, degp_ref, b_ref, out_ref):
    deg = degp_ref[0, :] + degp_ref[1, :] + 1.0
    dis = lax.rsqrt(deg)
    s = (accp_ref[0] + accp_ref[1] + g_ref[...]) * dis[:, None]
    s = s + b_ref[...]
    out_ref[...] = 0.5 * s * (1.0 + lax.erf(s * 0.7071067811865476))


def kernel(x, edge_index, W, b):
    src = edge_index[0].astype(jnp.int32)
    dst = edge_index[1].astype(jnp.int32)
    pad = TOTCH_PAD * CHUNK - N_EDGES
    src_p = jnp.concatenate([src, jnp.zeros((pad,), jnp.int32)])
    dst_p = jnp.concatenate([dst, jnp.zeros((pad,), jnp.int32)])
    sidx = src_p.reshape(TOTCH_PAD, CHUNK)
    didx = dst_p.reshape(TOTCH_PAD, CHUNK)

    degp = _deg_kernel(didx)

    xp = jnp.pad(x, ((0, NPAD - N_NODES), (0, 0)))

    g = pl.pallas_call(
        _lin_body,
        grid=(NPAD // BLK,),
        in_specs=[
            pl.BlockSpec((BLK, D), lambda i: (i, 0)),
            pl.BlockSpec((D, D), lambda i: (0, 0)),
            pl.BlockSpec((NC, BLK), lambda i: (0, i)),
        ],
        out_specs=pl.BlockSpec((BLK, D), lambda i: (i, 0)),
        out_shape=jax.ShapeDtypeStruct((NPAD, D), jnp.float32),
    )(xp, W, degp)

    accp = _scatter_kernel(g, sidx, didx)

    out = pl.pallas_call(
        _final_body,
        grid=(NPAD // BLK,),
        in_specs=[
            pl.BlockSpec((NC, BLK, D), lambda i: (0, i, 0)),
            pl.BlockSpec((BLK, D), lambda i: (i, 0)),
            pl.BlockSpec((NC, BLK), lambda i: (0, i)),
            pl.BlockSpec((1, D), lambda i: (0, 0)),
        ],
        out_specs=pl.BlockSpec((BLK, D), lambda i: (i, 0)),
        out_shape=jax.ShapeDtypeStruct((NPAD, D), jnp.float32),
    )(accp, g, degp, b.reshape(1, D))

    return out[:N_NODES]


# speedup vs baseline: 1.1111x; 1.1111x over previous
import functools

import jax
import jax.numpy as jnp
from jax import lax
from jax.experimental import pallas as pl
from jax.experimental.pallas import tpu as pltpu
from jax.experimental.pallas import tpu_sc as plsc

NC = 2
NS = 16
NW = NC * NS
CHUNK = 128

N_NODES = 10000
N_EDGES = 320000
D = 128

TOTCH = N_EDGES // CHUNK
TOTCH_PAD = 2560
NPAD = 10240
RPT = NPAD // NS
ACC_CH = NPAD // CHUNK
ACC_CPT = ACC_CH // NS

T0_CHUNKS = 1408
CPT0 = T0_CHUNKS // NS
CPT1 = 72

IBLK = 8

DSLAB = 80

_sc_mesh = plsc.VectorSubcoreMesh(core_axis_name="c", subcore_axis_name="s")


def _deg_body(didx_hbm, out_hbm, didx_v, ones_v, zbuf_v, deg_sh, sem):
    cid = lax.axis_index("c")
    sid = lax.axis_index("s")
    wid = sid * NC + cid
    base = wid * DSLAB
    nch = jnp.minimum(DSLAB, TOTCH - base)

    def fill16(i, _):
        ones_v[pl.ds(i * 16, 16)] = jnp.full((16,), 1.0, jnp.float32)
        return ()

    lax.fori_loop(0, CHUNK // 16, fill16, ())

    def zfill(i, _):
        zbuf_v[pl.ds(i * 16, 16)] = jnp.zeros((16,), jnp.float32)
        return ()

    lax.fori_loop(0, RPT // 16, zfill, ())

    pltpu.sync_copy(zbuf_v, deg_sh.at[pl.ds(sid * RPT, RPT)])

    pltpu.sync_copy(didx_hbm.at[pl.ds(base, DSLAB)], didx_v)
    plsc.subcore_barrier()

    def fire(j, _):
        pltpu.async_copy(ones_v, deg_sh.at[didx_v.at[j]], sem, add=True)
        return ()

    lax.fori_loop(0, nch, fire, ())

    def drain(j, _):
        pltpu.make_async_copy(ones_v, deg_sh.at[didx_v.at[0]], sem).wait()
        return ()

    lax.fori_loop(0, nch, drain, ())
    plsc.subcore_barrier()

    pltpu.sync_copy(deg_sh.at[pl.ds(sid * RPT, RPT)], zbuf_v)
    pltpu.sync_copy(zbuf_v, out_hbm.at[cid, pl.ds(sid * RPT, RPT)])


_deg_kernel = functools.partial(
    pl.kernel,
    out_type=jax.ShapeDtypeStruct((NC, NPAD), jnp.float32),
    mesh=_sc_mesh,
    scratch_types=[
        pltpu.VMEM((DSLAB, CHUNK), jnp.int32),
        pltpu.VMEM((CHUNK,), jnp.float32),
        pltpu.VMEM((RPT,), jnp.float32),
        pltpu.VMEM_SHARED((NPAD,), jnp.float32),
        pltpu.SemaphoreType.DMA,
    ],
)(_deg_body)


def _scatter_body(g_hbm, sidx_hbm, didx_hbm, out_hbm,
                  sidx_v, didx_v, bufs, acc_sh, semg, sems, semi):
    cid = lax.axis_index("c")
    sid = lax.axis_index("s")
    base = jnp.where(cid == 0, sid * CPT0, T0_CHUNKS + sid * CPT1)
    nch = jnp.where(
        cid == 0,
        CPT0,
        jnp.clip(TOTCH - base, 0, CPT1),
    )

    def zfill(i, _):
        r = i // (D // 16)
        c = i % (D // 16)
        bufs[0, r, pl.ds(c * 16, 16)] = jnp.zeros((16,), jnp.float32)
        return ()

    lax.fori_loop(0, CHUNK * (D // 16), zfill, ())

    for m in range(ACC_CPT):
        ch = sid + NS * m
        pltpu.async_copy(
            bufs.at[0], acc_sh.at[pl.ds(ch * CHUNK, CHUNK)], semi)

    pltpu.sync_copy(sidx_hbm.at[pl.ds(base, IBLK)], sidx_v.at[0])
    pltpu.sync_copy(didx_hbm.at[pl.ds(base, IBLK)], didx_v.at[0])
    for m in range(ACC_CPT):
        pltpu.make_async_copy(
            bufs.at[0], acc_sh.at[pl.ds(0, CHUNK)], semi).wait()
    plsc.subcore_barrier()

    @pl.when(nch > 0)
    def _():
        pltpu.async_copy(g_hbm.at[sidx_v.at[0, 0]], bufs.at[0], semg.at[0])

    def step(j, _):
        par = j % 2
        npar = (j + 1) % 2
        blk = j // IBLK
        pos = j % IBLK

        @pl.when(j >= 1)
        def _():
            pltpu.make_async_copy(
                bufs.at[0], acc_sh.at[pl.ds(0, CHUNK)], sems.at[npar]
            ).wait()

        @pl.when((pos == 1) & ((blk + 1) * IBLK < nch))
        def _():
            pltpu.async_copy(
                sidx_hbm.at[pl.ds(base + (blk + 1) * IBLK, IBLK)],
                sidx_v.at[(blk + 1) % 2], semi)
            pltpu.async_copy(
                didx_hbm.at[pl.ds(base + (blk + 1) * IBLK, IBLK)],
                didx_v.at[(blk + 1) % 2], semi)

        @pl.when(j + 1 < nch)
        def _():
            @pl.when(pos == IBLK - 1)
            def _():
                pltpu.make_async_copy(
                    sidx_hbm.at[pl.ds(base, IBLK)], sidx_v.at[0], semi
                ).wait()
                pltpu.make_async_copy(
                    sidx_hbm.at[pl.ds(base, IBLK)], sidx_v.at[0], semi
                ).wait()

            pltpu.async_copy(
                g_hbm.at[sidx_v.at[((j + 1) // IBLK) % 2, (j + 1) % IBLK]],
                bufs.at[npar], semg.at[npar])

        pltpu.make_async_copy(
            g_hbm.at[sidx_v.at[0, 0]], bufs.at[par], semg.at[par]
        ).wait()
        pltpu.async_copy(
            bufs.at[par], acc_sh.at[didx_v.at[blk % 2, pos]],
            sems.at[par], add=True)
        return ()

    lax.fori_loop(0, nch, step, ())

    @pl.when(nch > 0)
    def _():
        pltpu.make_async_copy(
            bufs.at[0], acc_sh.at[pl.ds(0, CHUNK)], sems.at[(nch - 1) % 2]
        ).wait()

    plsc.subcore_barrier()

    for m in range(ACC_CPT):
        ch = sid + NS * m
        pltpu.async_copy(
            acc_sh.at[pl.ds(ch * CHUNK, CHUNK)],
            out_hbm.at[cid, pl.ds(ch * CHUNK, CHUNK)], semi)
    for m in range(ACC_CPT):
        pltpu.make_async_copy(
            acc_sh.at[pl.ds(0, CHUNK)],
            out_hbm.at[cid, pl.ds(0, CHUNK)], semi).wait()


_scatter_kernel = functools.partial(
    pl.kernel,
    out_type=jax.ShapeDtypeStruct((NC, NPAD, D), jnp.float32),
    mesh=_sc_mesh,
    scratch_types=[
        pltpu.VMEM((2, IBLK, CHUNK), jnp.int32),
        pltpu.VMEM((2, IBLK, CHUNK), jnp.int32),
        pltpu.VMEM((2, CHUNK, D), jnp.float32),
        pltpu.VMEM_SHARED((NPAD, D), jnp.float32),
        pltpu.SemaphoreType.DMA((2,)),
        pltpu.SemaphoreType.DMA((2,)),
        pltpu.SemaphoreType.DMA,
    ],
)(_scatter_body)


BLK = 1024


def _lin_body(x_ref, w_ref, degp_ref, g_ref):
    deg = degp_ref[0, :] + degp_ref[1, :] + 1.0
    dis = lax.rsqrt(deg)
    h = lax.dot_general(
        x_ref[...], w_ref[...],
        (((1,), (1,)), ((), ())),
        preferred_element_type=jnp.float32,
    )
    g_ref[...] = h * dis[:, None]


def _final_body(accp_ref, g_ref, degp_ref, b_ref, out_ref):
    deg = degp_ref[0, :] + degp_ref[1, :] + 1.0
    dis = lax.rsqrt(deg)
    s = (accp_ref[0] + accp_ref[1] + g_ref[...]) * dis[:, None]
    s = s + b_ref[...]
    out_ref[...] = 0.5 * s * (1.0 + lax.erf(s * 0.7071067811865476))


def kernel(x, edge_index, W, b):
    src = edge_index[0].astype(jnp.int32)
    dst = edge_index[1].astype(jnp.int32)
    pad = TOTCH_PAD * CHUNK - N_EDGES
    src_p = jnp.concatenate([src, jnp.zeros((pad,), jnp.int32)])
    dst_p = jnp.concatenate([dst, jnp.zeros((pad,), jnp.int32)])
    sidx = src_p.reshape(TOTCH_PAD, CHUNK)
    didx = dst_p.reshape(TOTCH_PAD, CHUNK)

    degp = _deg_kernel(didx)

    xp = jnp.pad(x, ((0, NPAD - N_NODES), (0, 0)))

    g = pl.pallas_call(
        _lin_body,
        grid=(NPAD // BLK,),
        in_specs=[
            pl.BlockSpec((BLK, D), lambda i: (i, 0)),
            pl.BlockSpec((D, D), lambda i: (0, 0)),
            pl.BlockSpec((NC, BLK), lambda i: (0, i)),
        ],
        out_specs=pl.BlockSpec((BLK, D), lambda i: (i, 0)),
        out_shape=jax.ShapeDtypeStruct((NPAD, D), jnp.float32),
    )(xp, W, degp)

    accp = _scatter_kernel(g, sidx, didx)

    out = pl.pallas_call(
        _final_body,
        grid=(NPAD // BLK,),
        in_specs=[
            pl.BlockSpec((NC, BLK, D), lambda i: (0, i, 0)),
            pl.BlockSpec((BLK, D), lambda i: (i, 0)),
            pl.BlockSpec((NC, BLK), lambda i: (0, i)),
            pl.BlockSpec((1, D), lambda i: (0, 0)),
        ],
        out_specs=pl.BlockSpec((BLK, D), lambda i: (i, 0)),
        out_shape=jax.ShapeDtypeStruct((NPAD, D), jnp.float32),
    )(accp, g, degp, b.reshape(1, D))

    return out[:N_NODES]


# speedup vs baseline: 1.1683x; 1.0515x over previous
import functools

import jax
import jax.numpy as jnp
from jax import lax
from jax.experimental import pallas as pl
from jax.experimental.pallas import tpu as pltpu
from jax.experimental.pallas import tpu_sc as plsc

NC = 2
NS = 16
NW = NC * NS
CHUNK = 128

N_NODES = 10000
N_EDGES = 320000
D = 128

TOTCH = N_EDGES // CHUNK
TOTCH_PAD = 2560
NPAD = 10240
RPT = NPAD // NS
ACC_CH = NPAD // CHUNK
ACC_CPT = ACC_CH // NS

T0_CHUNKS = 1280
CPT0 = T0_CHUNKS // NS
CPT1 = 80

IBLK = 8

DSLAB = 80

_sc_mesh = plsc.VectorSubcoreMesh(core_axis_name="c", subcore_axis_name="s")


def _deg_body(didx_hbm, out_hbm, didx_v, ones_v, zbuf_v, deg_sh, sem):
    cid = lax.axis_index("c")
    sid = lax.axis_index("s")
    wid = sid * NC + cid
    base = wid * DSLAB
    nch = jnp.minimum(DSLAB, TOTCH - base)

    def fill16(i, _):
        ones_v[pl.ds(i * 16, 16)] = jnp.full((16,), 1.0, jnp.float32)
        return ()

    lax.fori_loop(0, CHUNK // 16, fill16, ())

    def zfill(i, _):
        zbuf_v[pl.ds(i * 16, 16)] = jnp.zeros((16,), jnp.float32)
        return ()

    lax.fori_loop(0, RPT // 16, zfill, ())

    pltpu.sync_copy(zbuf_v, deg_sh.at[pl.ds(sid * RPT, RPT)])

    pltpu.sync_copy(didx_hbm.at[pl.ds(base, DSLAB)], didx_v)
    plsc.subcore_barrier()

    def fire(j, _):
        pltpu.async_copy(ones_v, deg_sh.at[didx_v.at[j]], sem, add=True)
        return ()

    lax.fori_loop(0, nch, fire, ())

    def drain(j, _):
        pltpu.make_async_copy(ones_v, deg_sh.at[didx_v.at[0]], sem).wait()
        return ()

    lax.fori_loop(0, nch, drain, ())
    plsc.subcore_barrier()

    pltpu.sync_copy(deg_sh.at[pl.ds(sid * RPT, RPT)], zbuf_v)
    pltpu.sync_copy(zbuf_v, out_hbm.at[cid, pl.ds(sid * RPT, RPT)])


_deg_kernel = functools.partial(
    pl.kernel,
    out_type=jax.ShapeDtypeStruct((NC, NPAD), jnp.float32),
    mesh=_sc_mesh,
    scratch_types=[
        pltpu.VMEM((DSLAB, CHUNK), jnp.int32),
        pltpu.VMEM((CHUNK,), jnp.float32),
        pltpu.VMEM((RPT,), jnp.float32),
        pltpu.VMEM_SHARED((NPAD,), jnp.float32),
        pltpu.SemaphoreType.DMA,
    ],
)(_deg_body)


def _scatter_body(g_hbm, sidx_hbm, didx_hbm, out_hbm,
                  sidx_v, didx_v, bufs, acc_sh, semg, sems, semi):
    cid = lax.axis_index("c")
    sid = lax.axis_index("s")
    base = jnp.where(cid == 0, sid * CPT0, T0_CHUNKS + sid * CPT1)
    nch = jnp.where(
        cid == 0,
        CPT0,
        jnp.clip(TOTCH - base, 0, CPT1),
    )

    def zfill(i, _):
        r = i // (D // 16)
        c = i % (D // 16)
        bufs[0, r, pl.ds(c * 16, 16)] = jnp.zeros((16,), jnp.float32)
        return ()

    lax.fori_loop(0, CHUNK * (D // 16), zfill, ())

    for m in range(ACC_CPT):
        ch = sid + NS * m
        pltpu.async_copy(
            bufs.at[0], acc_sh.at[pl.ds(ch * CHUNK, CHUNK)], semi)

    pltpu.sync_copy(sidx_hbm.at[pl.ds(base, IBLK)], sidx_v.at[0])
    pltpu.sync_copy(didx_hbm.at[pl.ds(base, IBLK)], didx_v.at[0])
    for m in range(ACC_CPT):
        pltpu.make_async_copy(
            bufs.at[0], acc_sh.at[pl.ds(0, CHUNK)], semi).wait()
    plsc.subcore_barrier()

    @pl.when(nch > 0)
    def _():
        pltpu.async_copy(g_hbm.at[sidx_v.at[0, 0]], bufs.at[0], semg.at[0])

    def step(j, _):
        par = j % 2
        npar = (j + 1) % 2
        blk = j // IBLK
        pos = j % IBLK

        @pl.when(j >= 1)
        def _():
            pltpu.make_async_copy(
                bufs.at[0], acc_sh.at[pl.ds(0, CHUNK)], sems.at[npar]
            ).wait()

        @pl.when((pos == 1) & ((blk + 1) * IBLK < nch))
        def _():
            pltpu.async_copy(
                sidx_hbm.at[pl.ds(base + (blk + 1) * IBLK, IBLK)],
                sidx_v.at[(blk + 1) % 2], semi)
            pltpu.async_copy(
                didx_hbm.at[pl.ds(base + (blk + 1) * IBLK, IBLK)],
                didx_v.at[(blk + 1) % 2], semi)

        @pl.when(j + 1 < nch)
        def _():
            @pl.when(pos == IBLK - 1)
            def _():
                pltpu.make_async_copy(
                    sidx_hbm.at[pl.ds(base, IBLK)], sidx_v.at[0], semi
                ).wait()
                pltpu.make_async_copy(
                    sidx_hbm.at[pl.ds(base, IBLK)], sidx_v.at[0], semi
                ).wait()

            pltpu.async_copy(
                g_hbm.at[sidx_v.at[((j + 1) // IBLK) % 2, (j + 1) % IBLK]],
                bufs.at[npar], semg.at[npar])

        pltpu.make_async_copy(
            g_hbm.at[sidx_v.at[0, 0]], bufs.at[par], semg.at[par]
        ).wait()
        pltpu.async_copy(
            bufs.at[par], acc_sh.at[didx_v.at[blk % 2, pos]],
            sems.at[par], add=True)
        return ()

    lax.fori_loop(0, nch, step, ())

    @pl.when(nch > 0)
    def _():
        pltpu.make_async_copy(
            bufs.at[0], acc_sh.at[pl.ds(0, CHUNK)], sems.at[(nch - 1) % 2]
        ).wait()

    plsc.subcore_barrier()

    for m in range(ACC_CPT):
        ch = sid + NS * m
        pltpu.async_copy(
            acc_sh.at[pl.ds(ch * CHUNK, CHUNK)],
            out_hbm.at[cid, pl.ds(ch * CHUNK, CHUNK)], semi)
    for m in range(ACC_CPT):
        pltpu.make_async_copy(
            acc_sh.at[pl.ds(0, CHUNK)],
            out_hbm.at[cid, pl.ds(0, CHUNK)], semi).wait()


_scatter_kernel = functools.partial(
    pl.kernel,
    out_type=jax.ShapeDtypeStruct((NC, NPAD, D), jnp.float32),
    mesh=_sc_mesh,
    scratch_types=[
        pltpu.VMEM((2, IBLK, CHUNK), jnp.int32),
        pltpu.VMEM((2, IBLK, CHUNK), jnp.int32),
        pltpu.VMEM((2, CHUNK, D), jnp.float32),
        pltpu.VMEM_SHARED((NPAD, D), jnp.float32),
        pltpu.SemaphoreType.DMA((2,)),
        pltpu.SemaphoreType.DMA((2,)),
        pltpu.SemaphoreType.DMA,
    ],
)(_scatter_body)


BLK = 1024


def _lin_body(x_ref, w_ref, degp_ref, g_ref):
    deg = degp_ref[0, :] + degp_ref[1, :] + 1.0
    dis = lax.rsqrt(deg)
    h = lax.dot_general(
        x_ref[...], w_ref[...],
        (((1,), (1,)), ((), ())),
        preferred_element_type=jnp.float32,
    )
    g_ref[...] = h * dis[:, None]


def _final_body(accp_ref, g_ref, degp_ref, b_ref, out_ref):
    deg = degp_ref[0, :] + degp_ref[1, :] + 1.0
    dis = lax.rsqrt(deg)
    s = (accp_ref[0] + accp_ref[1] + g_ref[...]) * dis[:, None]
    s = s + b_ref[...]
    out_ref[...] = 0.5 * s * (1.0 + lax.erf(s * 0.7071067811865476))


def kernel(x, edge_index, W, b):
    src = edge_index[0].astype(jnp.int32)
    dst = edge_index[1].astype(jnp.int32)
    pad = TOTCH_PAD * CHUNK - N_EDGES
    src_p = jnp.concatenate([src, jnp.zeros((pad,), jnp.int32)])
    dst_p = jnp.concatenate([dst, jnp.zeros((pad,), jnp.int32)])
    sidx = src_p.reshape(TOTCH_PAD, CHUNK)
    didx = dst_p.reshape(TOTCH_PAD, CHUNK)

    degp = _deg_kernel(didx)

    xp = jnp.pad(x, ((0, NPAD - N_NODES), (0, 0)))

    g = pl.pallas_call(
        _lin_body,
        grid=(NPAD // BLK,),
        in_specs=[
            pl.BlockSpec((BLK, D), lambda i: (i, 0)),
            pl.BlockSpec((D, D), lambda i: (0, 0)),
            pl.BlockSpec((NC, BLK), lambda i: (0, i)),
        ],
        out_specs=pl.BlockSpec((BLK, D), lambda i: (i, 0)),
        out_shape=jax.ShapeDtypeStruct((NPAD, D), jnp.float32),
    )(xp, W, degp)

    accp = _scatter_kernel(g, sidx, didx)

    out = pl.pallas_call(
        _final_body,
        grid=(NPAD // BLK,),
        in_specs=[
            pl.BlockSpec((NC, BLK, D), lambda i: (0, i, 0)),
            pl.BlockSpec((BLK, D), lambda i: (i, 0)),
            pl.BlockSpec((NC, BLK), lambda i: (0, i)),
            pl.BlockSpec((1, D), lambda i: (0, 0)),
        ],
        out_specs=pl.BlockSpec((BLK, D), lambda i: (i, 0)),
        out_shape=jax.ShapeDtypeStruct((NPAD, D), jnp.float32),
    )(accp, g, degp, b.reshape(1, D))

    return out[:N_NODES]


# speedup vs baseline: 1.1695x; 1.0010x over previous
import functools

import jax
import jax.numpy as jnp
from jax import lax
from jax.experimental import pallas as pl
from jax.experimental.pallas import tpu as pltpu
from jax.experimental.pallas import tpu_sc as plsc

NC = 2
NS = 16
NW = NC * NS
CHUNK = 128

N_NODES = 10000
N_EDGES = 320000
D = 128

TOTCH = N_EDGES // CHUNK
NPAD = 10240
RPT = NPAD // NS
ACC_CH = NPAD // CHUNK
ACC_CPT = ACC_CH // NS

T0_CHUNKS = 1280
CPT0 = T0_CHUNKS // NS
CPT1 = 80

IBLK = 8
TAIL_START = TOTCH - TOTCH % IBLK
TAIL_REM = TOTCH - TAIL_START

DSLAB = 80

_sc_mesh = plsc.VectorSubcoreMesh(core_axis_name="c", subcore_axis_name="s")


def _deg_body(didx_hbm, tail_hbm, out_hbm, didx_v, ones_v, zbuf_v, deg_sh,
              sem):
    cid = lax.axis_index("c")
    sid = lax.axis_index("s")
    wid = sid * NC + cid
    base = wid * DSLAB
    nch = jnp.minimum(DSLAB, TOTCH - base)

    def fill16(i, _):
        ones_v[pl.ds(i * 16, 16)] = jnp.full((16,), 1.0, jnp.float32)
        return ()

    lax.fori_loop(0, CHUNK // 16, fill16, ())

    def zfill(i, _):
        zbuf_v[pl.ds(i * 16, 16)] = jnp.zeros((16,), jnp.float32)
        return ()

    lax.fori_loop(0, RPT // 16, zfill, ())

    pltpu.sync_copy(zbuf_v, deg_sh.at[pl.ds(sid * RPT, RPT)])

    @pl.when(wid < NW - 1)
    def _():
        pltpu.sync_copy(didx_hbm.at[pl.ds(base, DSLAB)], didx_v)

    @pl.when(wid == NW - 1)
    def _():
        head = TAIL_START - (NW - 1) * DSLAB
        pltpu.sync_copy(
            didx_hbm.at[pl.ds(base, head)], didx_v.at[pl.ds(0, head)])
        pltpu.sync_copy(
            tail_hbm.at[1], didx_v.at[pl.ds(head, IBLK)])
    plsc.subcore_barrier()

    def fire(j, _):
        pltpu.async_copy(ones_v, deg_sh.at[didx_v.at[j]], sem, add=True)
        return ()

    lax.fori_loop(0, nch, fire, ())

    def drain(j, _):
        pltpu.make_async_copy(ones_v, deg_sh.at[didx_v.at[0]], sem).wait()
        return ()

    lax.fori_loop(0, nch, drain, ())
    plsc.subcore_barrier()

    pltpu.sync_copy(deg_sh.at[pl.ds(sid * RPT, RPT)], zbuf_v)
    pltpu.sync_copy(zbuf_v, out_hbm.at[cid, pl.ds(sid * RPT, RPT)])


_deg_kernel = functools.partial(
    pl.kernel,
    out_type=jax.ShapeDtypeStruct((NC, NPAD), jnp.float32),
    mesh=_sc_mesh,
    scratch_types=[
        pltpu.VMEM((DSLAB, CHUNK), jnp.int32),
        pltpu.VMEM((CHUNK,), jnp.float32),
        pltpu.VMEM((RPT,), jnp.float32),
        pltpu.VMEM_SHARED((NPAD,), jnp.float32),
        pltpu.SemaphoreType.DMA,
    ],
)(_deg_body)


def _scatter_body(g_hbm, sidx_hbm, didx_hbm, tail_hbm, out_hbm,
                  sidx_v, didx_v, bufs, acc_sh, semg, sems, semi):
    cid = lax.axis_index("c")
    sid = lax.axis_index("s")
    base = jnp.where(cid == 0, sid * CPT0, T0_CHUNKS + sid * CPT1)
    nch = jnp.where(
        cid == 0,
        CPT0,
        jnp.clip(TOTCH - base, 0, CPT1),
    )

    def zfill(i, _):
        r = i // (D // 16)
        c = i % (D // 16)
        bufs[0, r, pl.ds(c * 16, 16)] = jnp.zeros((16,), jnp.float32)
        return ()

    lax.fori_loop(0, CHUNK * (D // 16), zfill, ())

    for m in range(ACC_CPT):
        ch = sid + NS * m
        pltpu.async_copy(
            bufs.at[0], acc_sh.at[pl.ds(ch * CHUNK, CHUNK)], semi)

    pltpu.sync_copy(sidx_hbm.at[pl.ds(base, IBLK)], sidx_v.at[0])
    pltpu.sync_copy(didx_hbm.at[pl.ds(base, IBLK)], didx_v.at[0])
    for m in range(ACC_CPT):
        pltpu.make_async_copy(
            bufs.at[0], acc_sh.at[pl.ds(0, CHUNK)], semi).wait()
    plsc.subcore_barrier()

    @pl.when(nch > 0)
    def _():
        pltpu.async_copy(g_hbm.at[sidx_v.at[0, 0]], bufs.at[0], semg.at[0])

    def step(j, _):
        par = j % 2
        npar = (j + 1) % 2
        blk = j // IBLK
        pos = j % IBLK

        @pl.when(j >= 1)
        def _():
            pltpu.make_async_copy(
                bufs.at[0], acc_sh.at[pl.ds(0, CHUNK)], sems.at[npar]
            ).wait()

        nxt = base + (blk + 1) * IBLK

        @pl.when((pos == 1) & ((blk + 1) * IBLK < nch) & (nxt < TAIL_START))
        def _():
            pltpu.async_copy(
                sidx_hbm.at[pl.ds(nxt, IBLK)], sidx_v.at[(blk + 1) % 2],
                semi)
            pltpu.async_copy(
                didx_hbm.at[pl.ds(nxt, IBLK)], didx_v.at[(blk + 1) % 2],
                semi)

        @pl.when((pos == 1) & ((blk + 1) * IBLK < nch) & (nxt >= TAIL_START))
        def _():
            pltpu.async_copy(tail_hbm.at[0], sidx_v.at[(blk + 1) % 2], semi)
            pltpu.async_copy(tail_hbm.at[1], didx_v.at[(blk + 1) % 2], semi)

        @pl.when(j + 1 < nch)
        def _():
            @pl.when(pos == IBLK - 1)
            def _():
                pltpu.make_async_copy(
                    sidx_hbm.at[pl.ds(base, IBLK)], sidx_v.at[0], semi
                ).wait()
                pltpu.make_async_copy(
                    sidx_hbm.at[pl.ds(base, IBLK)], sidx_v.at[0], semi
                ).wait()

            pltpu.async_copy(
                g_hbm.at[sidx_v.at[((j + 1) // IBLK) % 2, (j + 1) % IBLK]],
                bufs.at[npar], semg.at[npar])

        pltpu.make_async_copy(
            g_hbm.at[sidx_v.at[0, 0]], bufs.at[par], semg.at[par]
        ).wait()
        pltpu.async_copy(
            bufs.at[par], acc_sh.at[didx_v.at[blk % 2, pos]],
            sems.at[par], add=True)
        return ()

    lax.fori_loop(0, nch, step, ())

    @pl.when(nch > 0)
    def _():
        pltpu.make_async_copy(
            bufs.at[0], acc_sh.at[pl.ds(0, CHUNK)], sems.at[(nch - 1) % 2]
        ).wait()

    plsc.subcore_barrier()

    for m in range(ACC_CPT):
        ch = sid + NS * m
        pltpu.async_copy(
            acc_sh.at[pl.ds(ch * CHUNK, CHUNK)],
            out_hbm.at[cid, pl.ds(ch * CHUNK, CHUNK)], semi)
    for m in range(ACC_CPT):
        pltpu.make_async_copy(
            acc_sh.at[pl.ds(0, CHUNK)],
            out_hbm.at[cid, pl.ds(0, CHUNK)], semi).wait()


_scatter_kernel = functools.partial(
    pl.kernel,
    out_type=jax.ShapeDtypeStruct((NC, NPAD, D), jnp.float32),
    mesh=_sc_mesh,
    scratch_types=[
        pltpu.VMEM((2, IBLK, CHUNK), jnp.int32),
        pltpu.VMEM((2, IBLK, CHUNK), jnp.int32),
        pltpu.VMEM((2, CHUNK, D), jnp.float32),
        pltpu.VMEM_SHARED((NPAD, D), jnp.float32),
        pltpu.SemaphoreType.DMA((2,)),
        pltpu.SemaphoreType.DMA((2,)),
        pltpu.SemaphoreType.DMA,
    ],
)(_scatter_body)


BLK = 1024


def _lin_body(x_ref, w_ref, degp_ref, g_ref):
    deg = degp_ref[0, :] + degp_ref[1, :] + 1.0
    dis = lax.rsqrt(deg)
    h = lax.dot_general(
        x_ref[...], w_ref[...],
        (((1,), (1,)), ((), ())),
        preferred_element_type=jnp.float32,
    )
    g_ref[...] = h * dis[:, None]


def _final_body(accp_ref, g_ref, degp_ref, b_ref, out_ref):
    deg = degp_ref[0, :] + degp_ref[1, :] + 1.0
    dis = lax.rsqrt(deg)
    s = (accp_ref[0] + accp_ref[1] + g_ref[...]) * dis[:, None]
    s = s + b_ref[...]
    out_ref[...] = 0.5 * s * (1.0 + lax.erf(s * 0.7071067811865476))


def kernel(x, edge_index, W, b):
    sidx = edge_index[0].astype(jnp.int32).reshape(TOTCH, CHUNK)
    didx = edge_index[1].astype(jnp.int32).reshape(TOTCH, CHUNK)
    tail = jnp.zeros((2, IBLK, CHUNK), jnp.int32)
    tail = tail.at[:, :TAIL_REM].set(
        jnp.stack([sidx[TAIL_START:], didx[TAIL_START:]]))

    degp = _deg_kernel(didx, tail)

    xp = jnp.pad(x, ((0, NPAD - N_NODES), (0, 0)))

    g = pl.pallas_call(
        _lin_body,
        grid=(NPAD // BLK,),
        in_specs=[
            pl.BlockSpec((BLK, D), lambda i: (i, 0)),
            pl.BlockSpec((D, D), lambda i: (0, 0)),
            pl.BlockSpec((NC, BLK), lambda i: (0, i)),
        ],
        out_specs=pl.BlockSpec((BLK, D), lambda i: (i, 0)),
        out_shape=jax.ShapeDtypeStruct((NPAD, D), jnp.float32),
    )(xp, W, degp)

    accp = _scatter_kernel(g, sidx, didx, tail)

    out = pl.pallas_call(
        _final_body,
        grid=(NPAD // BLK,),
        in_specs=[
            pl.BlockSpec((NC, BLK, D), lambda i: (0, i, 0)),
            pl.BlockSpec((BLK, D), lambda i: (i, 0)),
            pl.BlockSpec((NC, BLK), lambda i: (0, i)),
            pl.BlockSpec((1, D), lambda i: (0, 0)),
        ],
        out_specs=pl.BlockSpec((BLK, D), lambda i: (i, 0)),
        out_shape=jax.ShapeDtypeStruct((NPAD, D), jnp.float32),
    )(accp, g, degp, b.reshape(1, D))

    return out[:N_NODES]


# speedup vs baseline: 1.2034x; 1.0290x over previous
import functools

import jax
import jax.numpy as jnp
from jax import lax
from jax.experimental import pallas as pl
from jax.experimental.pallas import tpu as pltpu
from jax.experimental.pallas import tpu_sc as plsc

NC = 2
NS = 16
NW = NC * NS
CHUNK = 128

N_NODES = 10000
N_EDGES = 320000
D = 128

TOTCH = N_EDGES // CHUNK
NPAD = 10240
RPT = NPAD // NS
ACC_CH = NPAD // CHUNK
ACC_CPT = ACC_CH // NS

T0_CHUNKS = 1280
CPT0 = T0_CHUNKS // NS
CPT1 = 80

IBLK = 8
TAIL_START = TOTCH - TOTCH % IBLK
TAIL_REM = TOTCH - TAIL_START

DSLAB = 80

_sc_mesh = plsc.VectorSubcoreMesh(core_axis_name="c", subcore_axis_name="s")


def _deg_body(didx_hbm, tail_hbm, out_hbm, didx_v, ones_v, zbuf_v, deg_sh,
              sem):
    cid = lax.axis_index("c")
    sid = lax.axis_index("s")
    wid = sid * NC + cid
    base = wid * DSLAB
    nch = jnp.minimum(DSLAB, TOTCH - base)

    def fill16(i, _):
        ones_v[pl.ds(i * 16, 16)] = jnp.full((16,), 1.0, jnp.float32)
        return ()

    lax.fori_loop(0, CHUNK // 16, fill16, ())

    def zfill(i, _):
        zbuf_v[pl.ds(i * 16, 16)] = jnp.zeros((16,), jnp.float32)
        return ()

    lax.fori_loop(0, RPT // 16, zfill, ())

    pltpu.sync_copy(zbuf_v, deg_sh.at[pl.ds(sid * RPT, RPT)])

    @pl.when(wid < NW - 1)
    def _():
        pltpu.sync_copy(didx_hbm.at[pl.ds(base, DSLAB)], didx_v)

    @pl.when(wid == NW - 1)
    def _():
        head = TAIL_START - (NW - 1) * DSLAB
        pltpu.sync_copy(
            didx_hbm.at[pl.ds(base, head)], didx_v.at[pl.ds(0, head)])
        pltpu.sync_copy(
            tail_hbm.at[1], didx_v.at[pl.ds(head, IBLK)])
    plsc.subcore_barrier()

    def fire(j, _):
        pltpu.async_copy(ones_v, deg_sh.at[didx_v.at[j]], sem, add=True)
        return ()

    lax.fori_loop(0, nch, fire, ())

    def drain(j, _):
        pltpu.make_async_copy(ones_v, deg_sh.at[didx_v.at[0]], sem).wait()
        return ()

    lax.fori_loop(0, nch, drain, ())
    plsc.subcore_barrier()

    pltpu.sync_copy(deg_sh.at[pl.ds(sid * RPT, RPT)], zbuf_v)
    pltpu.sync_copy(zbuf_v, out_hbm.at[cid, pl.ds(sid * RPT, RPT)])


_deg_kernel = functools.partial(
    pl.kernel,
    out_type=jax.ShapeDtypeStruct((NC, NPAD), jnp.float32),
    mesh=_sc_mesh,
    scratch_types=[
        pltpu.VMEM((DSLAB, CHUNK), jnp.int32),
        pltpu.VMEM((CHUNK,), jnp.float32),
        pltpu.VMEM((RPT,), jnp.float32),
        pltpu.VMEM_SHARED((NPAD,), jnp.float32),
        pltpu.SemaphoreType.DMA,
    ],
)(_deg_body)


def _scatter_body(g_hbm, sidx_hbm, didx_hbm, tail_hbm, out_hbm,
                  sidx_v, didx_v, bufs, acc_sh, semg, sems, semi):
    cid = lax.axis_index("c")
    sid = lax.axis_index("s")
    base = jnp.where(cid == 0, sid * CPT0, T0_CHUNKS + sid * CPT1)
    nch = jnp.where(
        cid == 0,
        CPT0,
        jnp.clip(TOTCH - base, 0, CPT1),
    )

    def zfill(i, _):
        r = i // (D // 16)
        c = i % (D // 16)
        bufs[0, r, pl.ds(c * 16, 16)] = jnp.zeros((16,), jnp.float32)
        return ()

    lax.fori_loop(0, CHUNK * (D // 16), zfill, ())

    for m in range(ACC_CPT):
        ch = sid + NS * m
        pltpu.async_copy(
            bufs.at[0], acc_sh.at[pl.ds(ch * CHUNK, CHUNK)], semi)

    pltpu.sync_copy(sidx_hbm.at[pl.ds(base, IBLK)], sidx_v.at[0])
    pltpu.sync_copy(didx_hbm.at[pl.ds(base, IBLK)], didx_v.at[0])
    for m in range(ACC_CPT):
        pltpu.make_async_copy(
            bufs.at[0], acc_sh.at[pl.ds(0, CHUNK)], semi).wait()
    plsc.subcore_barrier()

    @pl.when(nch > 0)
    def _():
        pltpu.async_copy(g_hbm.at[sidx_v.at[0, 0]], bufs.at[0], semg.at[0])

    def step(j, _):
        par = j % 2
        npar = (j + 1) % 2
        blk = j // IBLK
        pos = j % IBLK

        @pl.when(j >= 1)
        def _():
            pltpu.make_async_copy(
                bufs.at[0], acc_sh.at[pl.ds(0, CHUNK)], sems.at[npar]
            ).wait()

        nxt = base + (blk + 1) * IBLK

        @pl.when((pos == 1) & ((blk + 1) * IBLK < nch) & (nxt < TAIL_START))
        def _():
            pltpu.async_copy(
                sidx_hbm.at[pl.ds(nxt, IBLK)], sidx_v.at[(blk + 1) % 2],
                semi)
            pltpu.async_copy(
                didx_hbm.at[pl.ds(nxt, IBLK)], didx_v.at[(blk + 1) % 2],
                semi)

        @pl.when((pos == 1) & ((blk + 1) * IBLK < nch) & (nxt >= TAIL_START))
        def _():
            pltpu.async_copy(tail_hbm.at[0], sidx_v.at[(blk + 1) % 2], semi)
            pltpu.async_copy(tail_hbm.at[1], didx_v.at[(blk + 1) % 2], semi)

        @pl.when(j + 1 < nch)
        def _():
            @pl.when(pos == IBLK - 1)
            def _():
                pltpu.make_async_copy(
                    sidx_hbm.at[pl.ds(base, IBLK)], sidx_v.at[0], semi
                ).wait()
                pltpu.make_async_copy(
                    sidx_hbm.at[pl.ds(base, IBLK)], sidx_v.at[0], semi
                ).wait()

            pltpu.async_copy(
                g_hbm.at[sidx_v.at[((j + 1) // IBLK) % 2, (j + 1) % IBLK]],
                bufs.at[npar], semg.at[npar])

        pltpu.make_async_copy(
            g_hbm.at[sidx_v.at[0, 0]], bufs.at[par], semg.at[par]
        ).wait()
        pltpu.async_copy(
            bufs.at[par], acc_sh.at[didx_v.at[blk % 2, pos]],
            sems.at[par], add=True)
        return ()

    lax.fori_loop(0, nch, step, ())

    @pl.when(nch > 0)
    def _():
        pltpu.make_async_copy(
            bufs.at[0], acc_sh.at[pl.ds(0, CHUNK)], sems.at[(nch - 1) % 2]
        ).wait()

    plsc.subcore_barrier()

    for m in range(ACC_CPT):
        ch = sid + NS * m
        pltpu.async_copy(
            acc_sh.at[pl.ds(ch * CHUNK, CHUNK)],
            out_hbm.at[cid, pl.ds(ch * CHUNK, CHUNK)], semi)
    for m in range(ACC_CPT):
        pltpu.make_async_copy(
            acc_sh.at[pl.ds(0, CHUNK)],
            out_hbm.at[cid, pl.ds(0, CHUNK)], semi).wait()


_scatter_kernel = functools.partial(
    pl.kernel,
    out_type=jax.ShapeDtypeStruct((NC, NPAD, D), jnp.float32),
    mesh=_sc_mesh,
    scratch_types=[
        pltpu.VMEM((2, IBLK, CHUNK), jnp.int32),
        pltpu.VMEM((2, IBLK, CHUNK), jnp.int32),
        pltpu.VMEM((2, CHUNK, D), jnp.float32),
        pltpu.VMEM_SHARED((NPAD, D), jnp.float32),
        pltpu.SemaphoreType.DMA((2,)),
        pltpu.SemaphoreType.DMA((2,)),
        pltpu.SemaphoreType.DMA,
    ],
)(_scatter_body)


BLK = 1024


def _lin_body(x_ref, w_ref, degp_ref, g_ref):
    deg = degp_ref[0, :] + degp_ref[1, :] + 1.0
    dis = lax.rsqrt(deg)
    h = lax.dot_general(
        x_ref[...], w_ref[...],
        (((1,), (1,)), ((), ())),
        preferred_element_type=jnp.float32,
    )
    g_ref[...] = h * dis[:, None]


def _final_body(accp_ref, g_ref, degp_ref, b_ref, out_ref):
    deg = degp_ref[0, :] + degp_ref[1, :] + 1.0
    dis = lax.rsqrt(deg)
    s = (accp_ref[0] + accp_ref[1] + g_ref[...]) * dis[:, None]
    s = s + b_ref[...]
    out_ref[...] = 0.5 * s * (1.0 + lax.erf(s * 0.7071067811865476))


def kernel(x, edge_index, W, b):
    sidx = edge_index[0].astype(jnp.int32).reshape(TOTCH, CHUNK)
    didx = edge_index[1].astype(jnp.int32).reshape(TOTCH, CHUNK)
    tail = jnp.zeros((2, IBLK, CHUNK), jnp.int32)
    tail = tail.at[:, :TAIL_REM].set(
        jnp.stack([sidx[TAIL_START:], didx[TAIL_START:]]))

    degp = _deg_kernel(didx, tail)

    xp = jnp.pad(x, ((0, NPAD - N_NODES), (0, 0)))

    g = pl.pallas_call(
        _lin_body,
        grid=(NPAD // BLK,),
        in_specs=[
            pl.BlockSpec((BLK, D), lambda i: (i, 0)),
            pl.BlockSpec((D, D), lambda i: (0, 0)),
            pl.BlockSpec((NC, BLK), lambda i: (0, i)),
        ],
        out_specs=pl.BlockSpec((BLK, D), lambda i: (i, 0)),
        out_shape=jax.ShapeDtypeStruct((NPAD, D), jnp.float32),
    )(xp, W, degp)

    accp = _scatter_kernel(g, sidx, didx, tail)

    out = pl.pallas_call(
        _final_body,
        grid=(NPAD // BLK,),
        in_specs=[
            pl.BlockSpec((NC, BLK, D), lambda i: (0, i, 0)),
            pl.BlockSpec((BLK, D), lambda i: (i, 0)),
            pl.BlockSpec((NC, BLK), lambda i: (0, i)),
            pl.BlockSpec((1, D), lambda i: (0, 0)),
        ],
        out_specs=pl.BlockSpec((BLK, D), lambda i: (i, 0)),
        out_shape=jax.ShapeDtypeStruct((N_NODES, D), jnp.float32),
    )(accp, g, degp, b.reshape(1, D))

    return out
